# baseline (speedup 1.0000x reference)
import functools

import jax
import jax.numpy as jnp
from jax import lax
from jax.experimental import pallas as pl
from jax.experimental.pallas import tpu as pltpu
from jax.experimental.pallas import tpu_sc as plsc

VOCAB = 100000
EMBED = 64
BATCH = 1024
CTX = 20

_NC = 2
_NS = 16
_NW = _NC * _NS
_BPW = BATCH // _NW
_IPW = _BPW * CTX


@functools.cache
def _make_gather_mean():
    mesh = plsc.VectorSubcoreMesh(core_axis_name="c", subcore_axis_name="s")

    @functools.partial(
        pl.kernel,
        mesh=mesh,
        out_type=jax.ShapeDtypeStruct((BATCH, EMBED), jnp.float32),
        scratch_types=[
            pltpu.VMEM((_IPW,), jnp.int32),
            pltpu.VMEM((_IPW, EMBED), jnp.float32),
            pltpu.VMEM((_BPW, EMBED), jnp.float32),
            pltpu.SemaphoreType.DMA,
        ],
        compiler_params=pltpu.CompilerParams(use_tc_tiling_on_sc=False),
    )
    def _gather_mean(ctx_hbm, table_hbm, out_hbm, idx_v, rows_v, acc_v, sem):
        wid = lax.axis_index("s") * _NC + lax.axis_index("c")
        base = wid * _IPW
        pltpu.sync_copy(ctx_hbm.at[pl.ds(base, _IPW)], idx_v)
        pltpu.async_copy(table_hbm.at[idx_v], rows_v, sem).wait()

        def body(b, carry):
            for c in range(EMBED // 16):
                acc = rows_v[b * CTX, pl.ds(c * 16, 16)]
                for t in range(1, CTX):
                    acc = acc + rows_v[b * CTX + t, pl.ds(c * 16, 16)]
                acc_v[b, pl.ds(c * 16, 16)] = acc * (1.0 / CTX)
            return carry

        lax.fori_loop(0, _BPW, body, 0)
        pltpu.sync_copy(acc_v, out_hbm.at[pl.ds(wid * _BPW, _BPW)])

    return _gather_mean


_BLOCK_V = 2048
_NVB = pl.cdiv(VOCAB, _BLOCK_V)
_VPAD = _NVB * _BLOCK_V
_TAIL = VOCAB - (_NVB - 1) * _BLOCK_V
_NBUF = 5


def _proj_body(x_ref, w_ref, b_ref, o_hbm, bufs, tail_buf, sems, tail_sem):
    i = pl.program_id(0)
    s = lax.rem(i, _NBUF)
    last = _NVB - 1

    @pl.when(i >= _NBUF)
    def _():
        for k in range(_NBUF):
            @pl.when(s == k)
            def _():
                pltpu.make_async_copy(
                    bufs.at[k],
                    o_hbm.at[:, pl.ds((i - _NBUF) * _BLOCK_V, _BLOCK_V)],
                    sems.at[k],
                ).wait()

    col0 = pl.multiple_of(i * _BLOCK_V, _BLOCK_V)
    val = (
        lax.dot_general(
            x_ref[...], w_ref[...],
            (((1,), (1,)), ((), ())),
            preferred_element_type=jnp.float32,
        )
        + b_ref[:, pl.ds(col0, _BLOCK_V)]
    )

    @pl.when(i != last)
    def _():
        for k in range(_NBUF):
            @pl.when(s == k)
            def _():
                bufs[k] = val
                pltpu.make_async_copy(
                    bufs.at[k],
                    o_hbm.at[:, pl.ds(i * _BLOCK_V, _BLOCK_V)],
                    sems.at[k],
                ).start()

    @pl.when(i == last)
    def _():
        tail_buf[...] = val[:, :_TAIL]
        pltpu.make_async_copy(
            tail_buf,
            o_hbm.at[:, pl.ds(last * _BLOCK_V, _TAIL)],
            tail_sem,
        ).start()
        for k in range(_NBUF - 1):
            j = last - (_NBUF - 1) + k
            pltpu.make_async_copy(
                bufs.at[j % _NBUF],
                o_hbm.at[:, pl.ds(j * _BLOCK_V, _BLOCK_V)],
                sems.at[j % _NBUF],
            ).wait()
        pltpu.make_async_copy(
            tail_buf,
            o_hbm.at[:, pl.ds(last * _BLOCK_V, _TAIL)],
            tail_sem,
        ).wait()


_proj = pl.pallas_call(
    _proj_body,
    grid=(_NVB,),
    in_specs=[
        pl.BlockSpec((BATCH, EMBED), lambda i: (0, 0)),
        pl.BlockSpec((_BLOCK_V, EMBED), lambda i: (i, 0)),
        pl.BlockSpec((1, _VPAD), lambda i: (0, 0)),
    ],
    out_specs=pl.BlockSpec(memory_space=pl.ANY),
    out_shape=jax.ShapeDtypeStruct((BATCH, VOCAB), jnp.float32),
    scratch_shapes=[
        pltpu.VMEM((_NBUF, BATCH, _BLOCK_V), jnp.float32),
        pltpu.VMEM((BATCH, _TAIL), jnp.float32),
        pltpu.SemaphoreType.DMA((_NBUF,)),
        pltpu.SemaphoreType.DMA,
    ],
)


def kernel(context, emb_table, lin_w, lin_b):
    ctx_flat = context.astype(jnp.int32).reshape(-1)
    cv = _make_gather_mean()(ctx_flat, emb_table)
    b_pad = jnp.pad(lin_b, (0, _VPAD - VOCAB)).reshape(1, _VPAD)
    return _proj(cv, lin_w, b_pad)


# speedup vs baseline: 1.0006x; 1.0006x over previous
import functools

import jax
import jax.numpy as jnp
from jax import lax
from jax.experimental import pallas as pl
from jax.experimental.pallas import tpu as pltpu
from jax.experimental.pallas import tpu_sc as plsc

VOCAB = 100000
EMBED = 64
BATCH = 1024
CTX = 20

_NC = 2
_NS = 16
_NW = _NC * _NS
_BPW = BATCH // _NW
_IPW = _BPW * CTX


@functools.cache
def _make_gather_mean():
    mesh = plsc.VectorSubcoreMesh(core_axis_name="c", subcore_axis_name="s")

    @functools.partial(
        pl.kernel,
        mesh=mesh,
        out_type=jax.ShapeDtypeStruct((BATCH, EMBED), jnp.float32),
        scratch_types=[
            pltpu.VMEM((_IPW,), jnp.int32),
            pltpu.VMEM((_IPW, EMBED), jnp.float32),
            pltpu.VMEM((_BPW, EMBED), jnp.float32),
            pltpu.SemaphoreType.DMA,
        ],
        compiler_params=pltpu.CompilerParams(use_tc_tiling_on_sc=False),
    )
    def _gather_mean(ctx_hbm, table_hbm, out_hbm, idx_v, rows_v, acc_v, sem):
        wid = lax.axis_index("s") * _NC + lax.axis_index("c")
        base = wid * _IPW
        pltpu.sync_copy(ctx_hbm.at[pl.ds(base, _IPW)], idx_v)
        pltpu.async_copy(table_hbm.at[idx_v], rows_v, sem).wait()

        def body(b, carry):
            for c in range(EMBED // 16):
                acc = rows_v[b * CTX, pl.ds(c * 16, 16)]
                for t in range(1, CTX):
                    acc = acc + rows_v[b * CTX + t, pl.ds(c * 16, 16)]
                acc_v[b, pl.ds(c * 16, 16)] = acc * (1.0 / CTX)
            return carry

        lax.fori_loop(0, _BPW, body, 0)
        pltpu.sync_copy(acc_v, out_hbm.at[pl.ds(wid * _BPW, _BPW)])

    return _gather_mean


_BLOCK_V = 2048
_NVB = pl.cdiv(VOCAB, _BLOCK_V)
_VPAD = _NVB * _BLOCK_V
_TAIL = VOCAB - (_NVB - 1) * _BLOCK_V
_NBUF = 5


def _proj_body(x_ref, w_ref, b_ref, o_hbm, bufs, tail_buf, sems, tail_sem):
    i = pl.program_id(0)
    s = lax.rem(i, _NBUF)
    last = _NVB - 1

    @pl.when(i >= _NBUF)
    def _():
        for k in range(_NBUF):
            @pl.when(s == k)
            def _():
                pltpu.make_async_copy(
                    bufs.at[k],
                    o_hbm.at[:, pl.ds((i - _NBUF) * _BLOCK_V, _BLOCK_V)],
                    sems.at[k],
                ).wait()

    col0 = pl.multiple_of(i * _BLOCK_V, _BLOCK_V)
    val = (
        lax.dot_general(
            x_ref[...], w_ref[...],
            (((1,), (1,)), ((), ())),
            preferred_element_type=jnp.float32,
        )
        + b_ref[:, pl.ds(col0, _BLOCK_V)]
    )

    @pl.when(i != last)
    def _():
        for k in range(_NBUF):
            @pl.when(s == k)
            def _():
                bufs[k] = val
                pltpu.async_copy(
                    bufs.at[k],
                    o_hbm.at[:, pl.ds(i * _BLOCK_V, _BLOCK_V)],
                    sems.at[k],
                    priority=k % 2,
                )

    @pl.when(i == last)
    def _():
        tail_buf[...] = val[:, :_TAIL]
        pltpu.make_async_copy(
            tail_buf,
            o_hbm.at[:, pl.ds(last * _BLOCK_V, _TAIL)],
            tail_sem,
        ).start()
        for k in range(_NBUF - 1):
            j = last - (_NBUF - 1) + k
            pltpu.make_async_copy(
                bufs.at[j % _NBUF],
                o_hbm.at[:, pl.ds(j * _BLOCK_V, _BLOCK_V)],
                sems.at[j % _NBUF],
            ).wait()
        pltpu.make_async_copy(
            tail_buf,
            o_hbm.at[:, pl.ds(last * _BLOCK_V, _TAIL)],
            tail_sem,
        ).wait()


_proj = pl.pallas_call(
    _proj_body,
    grid=(_NVB,),
    in_specs=[
        pl.BlockSpec((BATCH, EMBED), lambda i: (0, 0)),
        pl.BlockSpec((_BLOCK_V, EMBED), lambda i: (i, 0)),
        pl.BlockSpec((1, _VPAD), lambda i: (0, 0)),
    ],
    out_specs=pl.BlockSpec(memory_space=pl.ANY),
    out_shape=jax.ShapeDtypeStruct((BATCH, VOCAB), jnp.float32),
    scratch_shapes=[
        pltpu.VMEM((_NBUF, BATCH, _BLOCK_V), jnp.float32),
        pltpu.VMEM((BATCH, _TAIL), jnp.float32),
        pltpu.SemaphoreType.DMA((_NBUF,)),
        pltpu.SemaphoreType.DMA,
    ],
)


def kernel(context, emb_table, lin_w, lin_b):
    ctx_flat = context.astype(jnp.int32).reshape(-1)
    cv = _make_gather_mean()(ctx_flat, emb_table)
    b_pad = jnp.pad(lin_b, (0, _VPAD - VOCAB)).reshape(1, _VPAD)
    return _proj(cv, lin_w, b_pad)


# speedup vs baseline: 1.9078x; 1.9066x over previous
import functools

import jax
import jax.numpy as jnp
from jax import lax
from jax.experimental import pallas as pl
from jax.experimental.pallas import tpu as pltpu
from jax.experimental.pallas import tpu_sc as plsc

VOCAB = 100000
EMBED = 64
BATCH = 1024
CTX = 20

_NC = 2
_NS = 16
_NW = _NC * _NS
_BPW = BATCH // _NW
_IPW = _BPW * CTX


@functools.cache
def _make_gather_mean():
    mesh = plsc.VectorSubcoreMesh(core_axis_name="c", subcore_axis_name="s")

    @functools.partial(
        pl.kernel,
        mesh=mesh,
        out_type=jax.ShapeDtypeStruct((BATCH, EMBED), jnp.float32),
        scratch_types=[
            pltpu.VMEM((_IPW,), jnp.int32),
            pltpu.VMEM((_IPW, EMBED), jnp.float32),
            pltpu.VMEM((_BPW, EMBED), jnp.float32),
            pltpu.SemaphoreType.DMA,
        ],
        compiler_params=pltpu.CompilerParams(use_tc_tiling_on_sc=False),
    )
    def _gather_mean(ctx_hbm, table_hbm, out_hbm, idx_v, rows_v, acc_v, sem):
        wid = lax.axis_index("s") * _NC + lax.axis_index("c")
        base = wid * _IPW
        pltpu.sync_copy(ctx_hbm.at[pl.ds(base, _IPW)], idx_v)
        pltpu.async_copy(table_hbm.at[idx_v], rows_v, sem).wait()

        def body(b, carry):
            for c in range(EMBED // 16):
                acc = rows_v[b * CTX, pl.ds(c * 16, 16)]
                for t in range(1, CTX):
                    acc = acc + rows_v[b * CTX + t, pl.ds(c * 16, 16)]
                acc_v[b, pl.ds(c * 16, 16)] = acc * (1.0 / CTX)
            return carry

        lax.fori_loop(0, _BPW, body, 0)
        pltpu.sync_copy(acc_v, out_hbm.at[pl.ds(wid * _BPW, _BPW)])

    return _gather_mean


_BLOCK_V = 2048
_NVB = pl.cdiv(VOCAB, _BLOCK_V)


def _proj_body(x_ref, w_ref, b_ref, o_ref):
    o_ref[...] = (
        lax.dot_general(
            w_ref[...], x_ref[...],
            (((1,), (1,)), ((), ())),
            preferred_element_type=jnp.float32,
        )
        + b_ref[...]
    )


_proj = pl.pallas_call(
    _proj_body,
    grid=(_NVB,),
    in_specs=[
        pl.BlockSpec((BATCH, EMBED), lambda i: (0, 0)),
        pl.BlockSpec((_BLOCK_V, EMBED), lambda i: (i, 0)),
        pl.BlockSpec((_BLOCK_V, 1), lambda i: (i, 0)),
    ],
    out_specs=pl.BlockSpec((_BLOCK_V, BATCH), lambda i: (i, 0)),
    out_shape=jax.ShapeDtypeStruct((VOCAB, BATCH), jnp.float32),
    compiler_params=pltpu.CompilerParams(
        dimension_semantics=("parallel",),
    ),
)


def kernel(context, emb_table, lin_w, lin_b):
    ctx_flat = context.astype(jnp.int32).reshape(-1)
    cv = _make_gather_mean()(ctx_flat, emb_table)
    return _proj(cv, lin_w, lin_b.reshape(VOCAB, 1)).T


# speedup vs baseline: 2.2227x; 1.1651x over previous
import functools

import jax
import jax.numpy as jnp
from jax import lax
from jax.experimental import pallas as pl
from jax.experimental.pallas import tpu as pltpu
from jax.experimental.pallas import tpu_sc as plsc

VOCAB = 100000
EMBED = 64
BATCH = 1024
CTX = 20

_NC = 2
_NS = 16
_NW = _NC * _NS
_BPW = BATCH // _NW
_IPW = _BPW * CTX


@functools.cache
def _make_gather_mean():
    mesh = plsc.VectorSubcoreMesh(core_axis_name="c", subcore_axis_name="s")

    @functools.partial(
        pl.kernel,
        mesh=mesh,
        out_type=jax.ShapeDtypeStruct((BATCH, EMBED), jnp.float32),
        scratch_types=[
            pltpu.VMEM((_IPW,), jnp.int32),
            pltpu.VMEM((_IPW, EMBED), jnp.float32),
            pltpu.VMEM((_BPW, EMBED), jnp.float32),
            pltpu.SemaphoreType.DMA,
        ],
        compiler_params=pltpu.CompilerParams(use_tc_tiling_on_sc=False),
    )
    def _gather_mean(ctx_hbm, table_hbm, out_hbm, idx_v, rows_v, acc_v, sem):
        wid = lax.axis_index("s") * _NC + lax.axis_index("c")
        base = wid * _IPW
        pltpu.sync_copy(ctx_hbm.at[pl.ds(base, _IPW)], idx_v)
        pltpu.async_copy(table_hbm.at[idx_v], rows_v, sem).wait()

        def body(b, carry):
            for c in range(EMBED // 16):
                acc = rows_v[b * CTX, pl.ds(c * 16, 16)]
                for t in range(1, CTX):
                    acc = acc + rows_v[b * CTX + t, pl.ds(c * 16, 16)]
                acc_v[b, pl.ds(c * 16, 16)] = acc * (1.0 / CTX)
            return carry

        lax.fori_loop(0, _BPW, body, 0)
        pltpu.sync_copy(acc_v, out_hbm.at[pl.ds(wid * _BPW, _BPW)])

    return _gather_mean


_BLOCK_V = 2048
_NVB = pl.cdiv(VOCAB, _BLOCK_V)


def _proj_body(x_ref, w_ref, b_ref, o_ref):
    o_ref[...] = (
        lax.dot_general(
            w_ref[...], x_ref[...],
            (((0,), (1,)), ((), ())),
            preferred_element_type=jnp.float32,
        )
        + b_ref[...]
    )


_proj = pl.pallas_call(
    _proj_body,
    grid=(_NVB,),
    in_specs=[
        pl.BlockSpec((BATCH, EMBED), lambda i: (0, 0)),
        pl.BlockSpec((EMBED, _BLOCK_V), lambda i: (0, i)),
        pl.BlockSpec((_BLOCK_V, 1), lambda i: (i, 0)),
    ],
    out_specs=pl.BlockSpec((_BLOCK_V, BATCH), lambda i: (i, 0)),
    out_shape=jax.ShapeDtypeStruct((VOCAB, BATCH), jnp.float32),
    compiler_params=pltpu.CompilerParams(
        dimension_semantics=("parallel",),
    ),
)


def kernel(context, emb_table, lin_w, lin_b):
    ctx_flat = context.astype(jnp.int32).reshape(-1)
    cv = _make_gather_mean()(ctx_flat, emb_table)
    return _proj(cv, lin_w.T, lin_b.reshape(VOCAB, 1)).T


# speedup vs baseline: 2.2269x; 1.0019x over previous
import functools

import jax
import jax.numpy as jnp
from jax import lax
from jax.experimental import pallas as pl
from jax.experimental.pallas import tpu as pltpu
from jax.experimental.pallas import tpu_sc as plsc

VOCAB = 100000
EMBED = 64
BATCH = 1024
CTX = 20

_NC = 2
_NS = 16
_NW = _NC * _NS
_BPW = BATCH // _NW
_IPW = _BPW * CTX


@functools.cache
def _make_gather_mean():
    mesh = plsc.VectorSubcoreMesh(core_axis_name="c", subcore_axis_name="s")

    @functools.partial(
        pl.kernel,
        mesh=mesh,
        out_type=jax.ShapeDtypeStruct((BATCH, EMBED), jnp.float32),
        scratch_types=[
            pltpu.VMEM((_IPW,), jnp.int32),
            pltpu.VMEM((_IPW, EMBED), jnp.float32),
            pltpu.VMEM((_BPW, EMBED), jnp.float32),
            pltpu.SemaphoreType.DMA,
        ],
        compiler_params=pltpu.CompilerParams(use_tc_tiling_on_sc=False),
    )
    def _gather_mean(ctx_hbm, table_hbm, out_hbm, idx_v, rows_v, acc_v, sem):
        wid = lax.axis_index("s") * _NC + lax.axis_index("c")
        base = wid * _IPW
        pltpu.sync_copy(ctx_hbm.at[pl.ds(base, _IPW)], idx_v)
        pltpu.async_copy(table_hbm.at[idx_v], rows_v, sem).wait()

        def body(b, carry):
            for c in range(EMBED // 16):
                acc = rows_v[b * CTX, pl.ds(c * 16, 16)]
                for t in range(1, CTX):
                    acc = acc + rows_v[b * CTX + t, pl.ds(c * 16, 16)]
                acc_v[b, pl.ds(c * 16, 16)] = acc * (1.0 / CTX)
            return carry

        lax.fori_loop(0, _BPW, body, 0)
        pltpu.sync_copy(acc_v, out_hbm.at[pl.ds(wid * _BPW, _BPW)])

    return _gather_mean


_BLOCK_V = 2048
_NVB = pl.cdiv(VOCAB, _BLOCK_V)


_TAIL = VOCAB - (_NVB - 1) * _BLOCK_V
_NBUF = 4


def _proj_body(x_ref, w_ref, b_ref, o_hbm, bufs, sems):
    i = pl.program_id(0)
    s = lax.rem(i, _NBUF)
    last = _NVB - 1

    @pl.when(i >= _NBUF)
    def _():
        for k in range(_NBUF):
            @pl.when(s == k)
            def _():
                pltpu.make_async_copy(
                    bufs.at[k],
                    o_hbm.at[pl.ds((i - _NBUF) * _BLOCK_V, _BLOCK_V), :],
                    sems.at[k],
                ).wait()

    val = (
        lax.dot_general(
            w_ref[...], x_ref[...],
            (((0,), (1,)), ((), ())),
            preferred_element_type=jnp.float32,
        )
        + b_ref[...]
    )

    @pl.when(i != last)
    def _():
        for k in range(_NBUF):
            @pl.when(s == k)
            def _():
                bufs[k] = val
                pltpu.make_async_copy(
                    bufs.at[k],
                    o_hbm.at[pl.ds(i * _BLOCK_V, _BLOCK_V), :],
                    sems.at[k],
                ).start()

    @pl.when(i == last)
    def _():
        kl = last % _NBUF
        bufs[kl] = val
        pltpu.make_async_copy(
            bufs.at[kl, :_TAIL, :],
            o_hbm.at[pl.ds(last * _BLOCK_V, _TAIL), :],
            sems.at[kl],
        ).start()
        for k in range(_NBUF):
            j = last - (_NBUF - 1) + k
            rows = _TAIL if j == last else _BLOCK_V
            pltpu.make_async_copy(
                bufs.at[j % _NBUF, :rows, :],
                o_hbm.at[pl.ds(j * _BLOCK_V, rows), :],
                sems.at[j % _NBUF],
            ).wait()


_proj = pl.pallas_call(
    _proj_body,
    grid=(_NVB,),
    in_specs=[
        pl.BlockSpec((BATCH, EMBED), lambda i: (0, 0)),
        pl.BlockSpec((EMBED, _BLOCK_V), lambda i: (0, i)),
        pl.BlockSpec((_BLOCK_V, 1), lambda i: (i, 0)),
    ],
    out_specs=pl.BlockSpec(memory_space=pl.ANY),
    out_shape=jax.ShapeDtypeStruct((VOCAB, BATCH), jnp.float32),
    scratch_shapes=[
        pltpu.VMEM((_NBUF, _BLOCK_V, BATCH), jnp.float32),
        pltpu.SemaphoreType.DMA((_NBUF,)),
    ],
    compiler_params=pltpu.CompilerParams(
        dimension_semantics=("arbitrary",),
    ),
)


def kernel(context, emb_table, lin_w, lin_b):
    ctx_flat = context.astype(jnp.int32).reshape(-1)
    cv = _make_gather_mean()(ctx_flat, emb_table)
    return _proj(cv, lin_w.T, lin_b.reshape(VOCAB, 1)).T


# speedup vs baseline: 2.2305x; 1.0016x over previous
import functools

import jax
import jax.numpy as jnp
from jax import lax
from jax.experimental import pallas as pl
from jax.experimental.pallas import tpu as pltpu
from jax.experimental.pallas import tpu_sc as plsc

VOCAB = 100000
EMBED = 64
BATCH = 1024
CTX = 20

_NC = 2
_NS = 16
_NW = _NC * _NS
_BPW = BATCH // _NW
_IPW = _BPW * CTX


@functools.cache
def _make_gather_mean():
    mesh = plsc.VectorSubcoreMesh(core_axis_name="c", subcore_axis_name="s")

    @functools.partial(
        pl.kernel,
        mesh=mesh,
        out_type=jax.ShapeDtypeStruct((BATCH, EMBED), jnp.float32),
        scratch_types=[
            pltpu.VMEM((_IPW,), jnp.int32),
            pltpu.VMEM((_IPW, EMBED), jnp.float32),
            pltpu.VMEM((_BPW, EMBED), jnp.float32),
            pltpu.SemaphoreType.DMA,
        ],
        compiler_params=pltpu.CompilerParams(use_tc_tiling_on_sc=False),
    )
    def _gather_mean(ctx_hbm, table_hbm, out_hbm, idx_v, rows_v, acc_v, sem):
        wid = lax.axis_index("s") * _NC + lax.axis_index("c")
        base = wid * _IPW
        pltpu.sync_copy(ctx_hbm.at[pl.ds(base, _IPW)], idx_v)
        pltpu.async_copy(table_hbm.at[idx_v], rows_v, sem).wait()

        def body(b, carry):
            for c in range(EMBED // 16):
                acc = rows_v[b * CTX, pl.ds(c * 16, 16)]
                for t in range(1, CTX):
                    acc = acc + rows_v[b * CTX + t, pl.ds(c * 16, 16)]
                acc_v[b, pl.ds(c * 16, 16)] = acc * (1.0 / CTX)
            return carry

        lax.fori_loop(0, _BPW, body, 0)
        pltpu.sync_copy(acc_v, out_hbm.at[pl.ds(wid * _BPW, _BPW)])

    return _gather_mean


_BLOCK_V = 2048
_NVB = pl.cdiv(VOCAB, _BLOCK_V)


_TAIL = VOCAB - (_NVB - 1) * _BLOCK_V
_NBUF = 4


def _proj_body(x_ref, w_ref, b_ref, o_hbm, bufs, sems):
    i = pl.program_id(0)
    s = lax.rem(i, _NBUF)
    last = _NVB - 1

    @pl.when(i >= _NBUF)
    def _():
        pltpu.make_async_copy(
            bufs.at[s],
            o_hbm.at[pl.ds((i - _NBUF) * _BLOCK_V, _BLOCK_V), :],
            sems.at[s],
        ).wait()

    val = (
        lax.dot_general(
            w_ref[...], x_ref[...],
            (((0,), (1,)), ((), ())),
            preferred_element_type=jnp.float32,
        )
        + b_ref[...]
    )
    bufs[s] = val

    @pl.when(i != last)
    def _():
        pltpu.make_async_copy(
            bufs.at[s],
            o_hbm.at[pl.ds(i * _BLOCK_V, _BLOCK_V), :],
            sems.at[s],
        ).start()

    @pl.when(i == last)
    def _():
        kl = last % _NBUF
        pltpu.make_async_copy(
            bufs.at[kl, :_TAIL, :],
            o_hbm.at[pl.ds(last * _BLOCK_V, _TAIL), :],
            sems.at[kl],
        ).start()
        for k in range(_NBUF):
            j = last - (_NBUF - 1) + k
            rows = _TAIL if j == last else _BLOCK_V
            pltpu.make_async_copy(
                bufs.at[j % _NBUF, :rows, :],
                o_hbm.at[pl.ds(j * _BLOCK_V, rows), :],
                sems.at[j % _NBUF],
            ).wait()


_proj = pl.pallas_call(
    _proj_body,
    grid=(_NVB,),
    in_specs=[
        pl.BlockSpec((BATCH, EMBED), lambda i: (0, 0)),
        pl.BlockSpec((EMBED, _BLOCK_V), lambda i: (0, i)),
        pl.BlockSpec((_BLOCK_V, 1), lambda i: (i, 0)),
    ],
    out_specs=pl.BlockSpec(memory_space=pl.ANY),
    out_shape=jax.ShapeDtypeStruct((VOCAB, BATCH), jnp.float32),
    scratch_shapes=[
        pltpu.VMEM((_NBUF, _BLOCK_V, BATCH), jnp.float32),
        pltpu.SemaphoreType.DMA((_NBUF,)),
    ],
    compiler_params=pltpu.CompilerParams(
        dimension_semantics=("arbitrary",),
    ),
)


def kernel(context, emb_table, lin_w, lin_b):
    ctx_flat = context.astype(jnp.int32).reshape(-1)
    cv = _make_gather_mean()(ctx_flat, emb_table)
    return _proj(cv, lin_w.T, lin_b.reshape(VOCAB, 1)).T
